# baseline (speedup 1.0000x reference)
import functools

import jax
import jax.numpy as jnp
from jax.experimental import pallas as pl
from jax.experimental.pallas import tpu as pltpu

_Q = 1024
_D = 384
_M = 50000
_BM = 2048
_QC = 256
_K = 5
_IPAD = 2.0 ** 30


def _normalize(x):
    n = jnp.linalg.norm(x, axis=-1, keepdims=True)
    return x / jnp.clip(n, 1e-12)


def _dot_nt(a, b):
    return jax.lax.dot_general(
        a.astype(jnp.bfloat16), b.astype(jnp.bfloat16),
        (((1,), (1,)), ((), ())),
        preferred_element_type=jnp.float32)


def _topk_kernel(td, tio, md, mio, vals_out, idx_out, rv, ri):
    m_step = pl.program_id(0)
    nm = pl.num_programs(0)

    @pl.when(m_step == 0)
    def _init():
        rv[...] = jnp.full((_Q, 128), -jnp.inf, jnp.float32)
        ri[...] = jnp.full((_Q, 128), _IPAD, jnp.float32)

    mdn = md[...]
    mion = mio[...]

    lane = jax.lax.broadcasted_iota(jnp.int32, (_QC, _BM), 1).astype(jnp.float32)
    gcol = lane + (m_step * _BM)
    amask = jnp.where(gcol < float(_M), 0.0, -jnp.inf)
    w23 = jnp.float32(2.0) / jnp.float32(3.0)

    for qi in range(_Q // _QC):
        sl = slice(qi * _QC, (qi + 1) * _QC)
        s = (2.0 * _dot_nt(td[sl, :], mdn)
             + w23 * _dot_nt(tio[sl, :], mion)) + amask

        sx = jnp.concatenate([s, rv[sl, :]], axis=1)
        gx = jnp.concatenate([gcol, ri[sl, :]], axis=1)

        lane128 = jax.lax.broadcasted_iota(jnp.int32, (_QC, 128), 1)
        nvt = jnp.full((_QC, 128), -jnp.inf, jnp.float32)
        nit = jnp.full((_QC, 128), _IPAD, jnp.float32)
        for k in range(_K):
            mval = jnp.max(sx, axis=1, keepdims=True)
            midx = jnp.min(jnp.where(sx == mval, gx, jnp.inf),
                           axis=1, keepdims=True)
            nvt = jnp.where(lane128 == k, mval, nvt)
            nit = jnp.where(lane128 == k, midx, nit)
            sx = jnp.where(gx == midx, -jnp.inf, sx)
        rv[sl, :] = nvt
        ri[sl, :] = nit

    @pl.when(m_step == nm - 1)
    def _emit():
        vals_out[...] = rv[...][:, 0:_K]
        idx_out[...] = ri[...][:, 0:_K].astype(jnp.int32)


@jax.jit
def _run(td, tio, md, mio):
    nm = pl.cdiv(_M, _BM)
    return pl.pallas_call(
        _topk_kernel,
        grid=(nm,),
        in_specs=[
            pl.BlockSpec((_Q, _D), lambda m: (0, 0)),
            pl.BlockSpec((_Q, 2 * _D), lambda m: (0, 0)),
            pl.BlockSpec((_BM, _D), lambda m: (m, 0)),
            pl.BlockSpec((_BM, 2 * _D), lambda m: (m, 0)),
        ],
        out_specs=[
            pl.BlockSpec((_Q, _K), lambda m: (0, 0)),
            pl.BlockSpec((_Q, _K), lambda m: (0, 0)),
        ],
        out_shape=[
            jax.ShapeDtypeStruct((_Q, _K), jnp.float32),
            jax.ShapeDtypeStruct((_Q, _K), jnp.int32),
        ],
        scratch_shapes=[
            pltpu.VMEM((_Q, 128), jnp.float32),
            pltpu.VMEM((_Q, 128), jnp.float32),
        ],
    )(td, tio, md, mio)


def kernel(task_desc, task_in, task_out, model_desc, model_in, model_out, top_k):
    td = _normalize(task_desc)
    tio = jnp.concatenate([_normalize(task_in), _normalize(task_out)], axis=1)
    md = _normalize(model_desc)
    mio = jnp.concatenate([_normalize(model_in), _normalize(model_out)], axis=1)
    vals, idx = _run(td, tio, md, mio)
    return vals, idx


# speedup vs baseline: 1.1726x; 1.1726x over previous
import functools

import jax
import jax.numpy as jnp
from jax.experimental import pallas as pl
from jax.experimental.pallas import tpu as pltpu

_Q = 1024
_D = 384
_M = 50000
_BM = 2048
_QC = 256
_K = 5
_IPAD = 2.0 ** 30


def _normalize(x):
    n = jnp.linalg.norm(x, axis=-1, keepdims=True)
    return x / jnp.clip(n, 1e-12)


def _dot_nt(a, b):
    return jax.lax.dot_general(
        a.astype(jnp.bfloat16), b.astype(jnp.bfloat16),
        (((1,), (1,)), ((), ())),
        preferred_element_type=jnp.float32)


def _topk_kernel(td, tio, md, mi, mo, vals_out, idx_out, rv, ri):
    m_step = pl.program_id(0)
    nm = pl.num_programs(0)

    @pl.when(m_step == 0)
    def _init():
        rv[...] = jnp.full((_Q, 128), -jnp.inf, jnp.float32)
        ri[...] = jnp.full((_Q, 128), _IPAD, jnp.float32)

    mdn = md[...]
    mion = jnp.concatenate([mi[...], mo[...]], axis=1)

    lane = jax.lax.broadcasted_iota(jnp.int32, (_QC, _BM), 1).astype(jnp.float32)
    gcol = lane + (m_step * _BM)
    amask = jnp.where(gcol < float(_M), 0.0, -jnp.inf)
    w23 = jnp.float32(2.0) / jnp.float32(3.0)

    for qi in range(_Q // _QC):
        sl = slice(qi * _QC, (qi + 1) * _QC)
        s = (2.0 * _dot_nt(td[sl, :], mdn)
             + w23 * _dot_nt(tio[sl, :], mion)) + amask

        sx = jnp.concatenate([s, rv[sl, :]], axis=1)
        gx = jnp.concatenate([gcol, ri[sl, :]], axis=1)

        lane128 = jax.lax.broadcasted_iota(jnp.int32, (_QC, 128), 1)
        nvt = jnp.full((_QC, 128), -jnp.inf, jnp.float32)
        nit = jnp.full((_QC, 128), _IPAD, jnp.float32)
        for k in range(_K):
            mval = jnp.max(sx, axis=1, keepdims=True)
            midx = jnp.min(jnp.where(sx == mval, gx, jnp.inf),
                           axis=1, keepdims=True)
            nvt = jnp.where(lane128 == k, mval, nvt)
            nit = jnp.where(lane128 == k, midx, nit)
            sx = jnp.where(gx == midx, -jnp.inf, sx)
        rv[sl, :] = nvt
        ri[sl, :] = nit

    @pl.when(m_step == nm - 1)
    def _emit():
        vals_out[...] = rv[...][:, 0:_K]
        idx_out[...] = ri[...][:, 0:_K].astype(jnp.int32)


@jax.jit
def _run(td, tio, md, mi, mo):
    nm = pl.cdiv(_M, _BM)
    return pl.pallas_call(
        _topk_kernel,
        grid=(nm,),
        in_specs=[
            pl.BlockSpec((_Q, _D), lambda m: (0, 0)),
            pl.BlockSpec((_Q, 2 * _D), lambda m: (0, 0)),
            pl.BlockSpec((_BM, _D), lambda m: (m, 0)),
            pl.BlockSpec((_BM, _D), lambda m: (m, 0)),
            pl.BlockSpec((_BM, _D), lambda m: (m, 0)),
        ],
        out_specs=[
            pl.BlockSpec((_Q, _K), lambda m: (0, 0)),
            pl.BlockSpec((_Q, _K), lambda m: (0, 0)),
        ],
        out_shape=[
            jax.ShapeDtypeStruct((_Q, _K), jnp.float32),
            jax.ShapeDtypeStruct((_Q, _K), jnp.int32),
        ],
        scratch_shapes=[
            pltpu.VMEM((_Q, 128), jnp.float32),
            pltpu.VMEM((_Q, 128), jnp.float32),
        ],
    )(td, tio, md, mi, mo)


def kernel(task_desc, task_in, task_out, model_desc, model_in, model_out, top_k):
    td = _normalize(task_desc)
    tio = jnp.concatenate([_normalize(task_in), _normalize(task_out)], axis=1)
    md = _normalize(model_desc)
    mi = _normalize(model_in)
    mo = _normalize(model_out)
    vals, idx = _run(td, tio, md, mi, mo)
    return vals, idx
